# baseline (speedup 1.0000x reference)
import functools

import numpy as np
import jax
import jax.numpy as jnp
from jax import lax
from jax.experimental import pallas as pl
from jax.experimental.pallas import tpu as pltpu
from jax.experimental.pallas import tpu_sc as plsc

_VOCAB = 100000
_N_NEGS = 64
_EMB = 128
_NCTX = 20
_NSAM = _N_NEGS + 1
_NSAM_PAD = 80
_NC = 2
_NS = 16
_NW = _NC * _NS
_NBUF = 2
_C8 = _EMB // 16

_neg_cache = {}


def _neg_words(batch):
    if batch in _neg_cache:
        return jnp.asarray(_neg_cache[batch])
    try:
        with jax.ensure_compile_time_eval():
            arr = np.asarray(
                jax.random.randint(jax.random.key(12345), (batch, _N_NEGS), 1, _VOCAB)
            )
        _neg_cache[batch] = arr
        return jnp.asarray(arr)
    except Exception:
        return jax.random.randint(jax.random.key(12345), (batch, _N_NEGS), 1, _VOCAB)


def _sc_body(whdn, wvec, ddoc, neg_idx, word_idx, ctx_idx, fun_idx, out,
             neg_idx_v, ctx_idx_v, word_idx_v, fun_idx_v, fun_rows, word_rows,
             sam_rows0, sam_rows1, ctx_rows0, ctx_rows1, scores_v,
             sem_f, sem_w, sem_s0, sem_s1, sem_c0, sem_c1):
    bt = scores_v.shape[0]
    sam_rows = (sam_rows0, sam_rows1)
    ctx_rows = (ctx_rows0, ctx_rows1)
    sem_s = (sem_s0, sem_s1)
    sem_c = (sem_c0, sem_c1)
    wid = lax.axis_index("s") * _NC + lax.axis_index("c")
    base = wid * bt

    npair = bt // 2
    pltpu.sync_copy(neg_idx.at[pl.ds(wid * npair, npair)], neg_idx_v)
    pltpu.sync_copy(ctx_idx.at[pl.ds(wid * npair, npair)], ctx_idx_v)
    pltpu.sync_copy(word_idx.at[pl.ds(base, bt)], word_idx_v)
    pltpu.sync_copy(fun_idx.at[pl.ds(base, bt)], fun_idx_v)
    fcopy = pltpu.async_copy(ddoc.at[fun_idx_v], fun_rows, sem_f)
    wcopy = pltpu.async_copy(whdn.at[word_idx_v], word_rows, sem_w)

    def fire(pr, slot):
        pltpu.async_copy(whdn.at[neg_idx_v.at[pr]], sam_rows[slot], sem_s[slot])
        pltpu.async_copy(wvec.at[ctx_idx_v.at[pr]], ctx_rows[slot], sem_c[slot])

    def drain(slot):
        pltpu.make_async_copy(
            whdn.at[neg_idx_v.at[0]], sam_rows[slot], sem_s[slot]).wait()
        pltpu.make_async_copy(
            wvec.at[ctx_idx_v.at[0]], ctx_rows[slot], sem_c[slot]).wait()

    for slot in range(_NBUF):
        fire(slot, slot)
    fcopy.wait()
    wcopy.wait()

    lanes = lax.iota(jnp.int32, 16)

    def compute(e, slot, sub):
        p = [fun_rows[e, pl.ds(16 * c, 16)] for c in range(_C8)]
        for j in range(_NCTX):
            for c in range(_C8):
                p[c] = p[c] + ctx_rows[slot][sub * _NCTX + j, pl.ds(16 * c, 16)]
        inv = jnp.float32(1.0 / (_NCTX + 1))
        p = [x * inv for x in p]

        def acc_row(j):
            acc = sam_rows[slot][sub * _N_NEGS + j, pl.ds(0, 16)] * p[0]
            for c in range(1, _C8):
                acc = acc + sam_rows[slot][sub * _N_NEGS + j, pl.ds(16 * c, 16)] * p[c]
            return acc

        for grp in range(4):
            vec = jnp.zeros((16,), jnp.float32)
            for jj in range(16):
                vec = jnp.where(lanes == jj, jnp.sum(acc_row(grp * 16 + jj)), vec)
            scores_v[e, pl.ds(16 * grp, 16)] = -vec
        wacc = word_rows[e, pl.ds(0, 16)] * p[0]
        for c in range(1, _C8):
            wacc = wacc + word_rows[e, pl.ds(16 * c, 16)] * p[c]
        wvecs = jnp.where(lanes == 0, jnp.sum(wacc), jnp.float32(0.0))
        scores_v[e, pl.ds(64, 16)] = wvecs

    def body(i, carry):
        for slot in range(_NBUF):
            pr = i * _NBUF + slot
            drain(slot)
            for sub in range(2):
                compute(pr * 2 + sub, slot, sub)
            fire(jnp.minimum(pr + _NBUF, npair - 1), slot)
        return carry

    lax.fori_loop(0, npair // _NBUF, body, 0)
    for slot in range(_NBUF):
        drain(slot)
    pltpu.sync_copy(scores_v, out.at[pl.ds(base, bt)])


def _make_sc_scores(B):
    bt = B // _NW
    return pl.kernel(
        _sc_body,
        out_type=jax.ShapeDtypeStruct((B, _NSAM_PAD), jnp.float32),
        mesh=plsc.VectorSubcoreMesh(
            core_axis_name="c", subcore_axis_name="s",
            num_cores=_NC, num_subcores=_NS),
        compiler_params=pltpu.CompilerParams(needs_layout_passes=False),
        scratch_types=[
            pltpu.VMEM((bt // 2, 2 * _N_NEGS), jnp.int32),
            pltpu.VMEM((bt // 2, 2 * _NCTX), jnp.int32),
            pltpu.VMEM((bt,), jnp.int32),
            pltpu.VMEM((bt,), jnp.int32),
            pltpu.VMEM((bt, _EMB), jnp.float32),
            pltpu.VMEM((bt, _EMB), jnp.float32),
            pltpu.VMEM((2 * _N_NEGS, _EMB), jnp.float32),
            pltpu.VMEM((2 * _N_NEGS, _EMB), jnp.float32),
            pltpu.VMEM((2 * _NCTX, _EMB), jnp.float32),
            pltpu.VMEM((2 * _NCTX, _EMB), jnp.float32),
            pltpu.VMEM((bt, _NSAM_PAD), jnp.float32),
            pltpu.SemaphoreType.DMA,
            pltpu.SemaphoreType.DMA,
            pltpu.SemaphoreType.DMA,
            pltpu.SemaphoreType.DMA,
            pltpu.SemaphoreType.DMA,
            pltpu.SemaphoreType.DMA,
        ],
    )


def _neg_log_sig_sum(x):
    sim = jax.nn.sigmoid(x)
    masked = jnp.where(sim == 0.0, 1.0, sim)
    return -jnp.sum(jnp.log(masked))


def _loss_from_scores(scores_list, batch):
    def body(*refs):
        *s_refs, o_ref = refs
        total = jnp.float32(0.0)
        for s in s_refs:
            x = s[...]
            if x.shape[1] == _NSAM_PAD:
                x = x[:, :_NSAM]
            total = total + _neg_log_sig_sum(x)

        o_ref[...] = jnp.reshape(total / batch, (1, 1))

    out = pl.pallas_call(
        body,
        out_shape=jax.ShapeDtypeStruct((1, 1), jnp.float32),
    )(*scores_list)
    return out[0, 0]


def kernel(fun, word, context, W_hdn, W_vec, D_doc):
    B = word.shape[0]
    fun = fun.astype(jnp.int32)
    word = word.astype(jnp.int32)
    context = context.astype(jnp.int32)
    neg = _neg_words(B)

    neg2 = neg.reshape(B // 2, 2 * _N_NEGS)
    ctx2 = context.reshape(B // 2, 2 * _NCTX)
    scores = _make_sc_scores(B)(W_hdn, W_vec, D_doc, neg2, word, ctx2, fun)
    return _loss_from_scores([scores], B)


# speedup vs baseline: 1.0584x; 1.0584x over previous
import functools

import numpy as np
import jax
import jax.numpy as jnp
from jax import lax
from jax.experimental import pallas as pl
from jax.experimental.pallas import tpu as pltpu
from jax.experimental.pallas import tpu_sc as plsc

_VOCAB = 100000
_N_NEGS = 64
_EMB = 128
_NCTX = 20
_NSAM = _N_NEGS + 1
_NSAM_PAD = 80
_NC = 2
_NS = 16
_NW = _NC * _NS
_NBUF = 2
_C8 = _EMB // 16

_neg_cache = {}


def _neg_words(batch):
    if batch in _neg_cache:
        return jnp.asarray(_neg_cache[batch])
    try:
        with jax.ensure_compile_time_eval():
            arr = np.asarray(
                jax.random.randint(jax.random.key(12345), (batch, _N_NEGS), 1, _VOCAB)
            )
        _neg_cache[batch] = arr
        return jnp.asarray(arr)
    except Exception:
        return jax.random.randint(jax.random.key(12345), (batch, _N_NEGS), 1, _VOCAB)


def _sc_body(whdn, wvec, ddoc, neg_idx, word_idx, ctx_idx, fun_idx, out,
             neg_idx_v, ctx_idx_v, word_idx_v, fun_idx_v, fun_rows, word_rows,
             sam_rows0, sam_rows1, ctx_rows0, ctx_rows1, scores_v,
             sem_f, sem_w, sem_s0, sem_s1, sem_c0, sem_c1):
    bt = scores_v.shape[0]
    sam_rows = (sam_rows0, sam_rows1)
    ctx_rows = (ctx_rows0, ctx_rows1)
    sem_s = (sem_s0, sem_s1)
    sem_c = (sem_c0, sem_c1)
    wid = lax.axis_index("s") * _NC + lax.axis_index("c")
    base = wid * bt

    pltpu.sync_copy(neg_idx.at[pl.ds(base, bt)], neg_idx_v)
    pltpu.sync_copy(ctx_idx.at[pl.ds(base, bt)], ctx_idx_v)
    pltpu.sync_copy(word_idx.at[pl.ds(base, bt)], word_idx_v)
    pltpu.sync_copy(fun_idx.at[pl.ds(base, bt)], fun_idx_v)
    fcopy = pltpu.async_copy(ddoc.at[fun_idx_v], fun_rows, sem_f)
    wcopy = pltpu.async_copy(whdn.at[word_idx_v], word_rows, sem_w)

    def fire(e, slot):
        pltpu.async_copy(whdn.at[neg_idx_v.at[e]], sam_rows[slot], sem_s[slot])
        pltpu.async_copy(wvec.at[ctx_idx_v.at[e]], ctx_rows[slot], sem_c[slot])

    def drain(slot):
        pltpu.make_async_copy(
            whdn.at[neg_idx_v.at[0]], sam_rows[slot], sem_s[slot]).wait()
        pltpu.make_async_copy(
            wvec.at[ctx_idx_v.at[0]], ctx_rows[slot], sem_c[slot]).wait()

    for slot in range(_NBUF):
        fire(slot, slot)
    fcopy.wait()
    wcopy.wait()

    lanes = lax.iota(jnp.int32, 16)

    def compute(e, slot):
        p = [fun_rows[e, pl.ds(16 * c, 16)] for c in range(_C8)]
        for j in range(_NCTX):
            for c in range(_C8):
                p[c] = p[c] + ctx_rows[slot][j, pl.ds(16 * c, 16)]
        inv = jnp.float32(1.0 / (_NCTX + 1))
        p = [x * inv for x in p]

        def acc_row(j):
            acc = sam_rows[slot][j, pl.ds(0, 16)] * p[0]
            for c in range(1, _C8):
                acc = acc + sam_rows[slot][j, pl.ds(16 * c, 16)] * p[c]
            return acc

        for grp in range(4):
            vec = jnp.zeros((16,), jnp.float32)
            for jj in range(16):
                vec = jnp.where(lanes == jj, jnp.sum(acc_row(grp * 16 + jj)), vec)
            scores_v[e, pl.ds(16 * grp, 16)] = -vec
        wacc = word_rows[e, pl.ds(0, 16)] * p[0]
        for c in range(1, _C8):
            wacc = wacc + word_rows[e, pl.ds(16 * c, 16)] * p[c]
        wvecs = jnp.where(lanes == 0, jnp.sum(wacc), jnp.float32(0.0))
        scores_v[e, pl.ds(64, 16)] = wvecs

    def body(i, carry):
        for slot in range(_NBUF):
            e = i * _NBUF + slot
            drain(slot)
            compute(e, slot)
            fire(jnp.minimum(e + _NBUF, bt - 1), slot)
        return carry

    lax.fori_loop(0, bt // _NBUF, body, 0)
    for slot in range(_NBUF):
        drain(slot)
    pltpu.sync_copy(scores_v, out.at[pl.ds(base, bt)])


def _make_sc_scores(B):
    bt = B // _NW
    return pl.kernel(
        _sc_body,
        out_type=jax.ShapeDtypeStruct((B, _NSAM_PAD), jnp.float32),
        mesh=plsc.VectorSubcoreMesh(
            core_axis_name="c", subcore_axis_name="s",
            num_cores=_NC, num_subcores=_NS),
        compiler_params=pltpu.CompilerParams(needs_layout_passes=False),
        scratch_types=[
            pltpu.VMEM((bt, _N_NEGS), jnp.int32),
            pltpu.VMEM((bt, _NCTX), jnp.int32),
            pltpu.VMEM((bt,), jnp.int32),
            pltpu.VMEM((bt,), jnp.int32),
            pltpu.VMEM((bt, _EMB), jnp.float32),
            pltpu.VMEM((bt, _EMB), jnp.float32),
            pltpu.VMEM((_N_NEGS, _EMB), jnp.float32),
            pltpu.VMEM((_N_NEGS, _EMB), jnp.float32),
            pltpu.VMEM((_NCTX, _EMB), jnp.float32),
            pltpu.VMEM((_NCTX, _EMB), jnp.float32),
            pltpu.VMEM((bt, _NSAM_PAD), jnp.float32),
            pltpu.SemaphoreType.DMA,
            pltpu.SemaphoreType.DMA,
            pltpu.SemaphoreType.DMA,
            pltpu.SemaphoreType.DMA,
            pltpu.SemaphoreType.DMA,
            pltpu.SemaphoreType.DMA,
        ],
    )


def _neg_log_sig_sum(x):
    sim = jax.nn.sigmoid(x)
    masked = jnp.where(sim == 0.0, 1.0, sim)
    return -jnp.sum(jnp.log(masked))


def _loss_from_scores(scores_list, batch):
    def body(*refs):
        *s_refs, o_ref = refs
        total = jnp.float32(0.0)
        for s in s_refs:
            x = s[...]
            if x.shape[1] == _NSAM_PAD:
                x = x[:, :_NSAM]
            total = total + _neg_log_sig_sum(x)

        o_ref[...] = jnp.reshape(total / batch, (1, 1))

    out = pl.pallas_call(
        body,
        out_shape=jax.ShapeDtypeStruct((1, 1), jnp.float32),
    )(*scores_list)
    return out[0, 0]


def kernel(fun, word, context, W_hdn, W_vec, D_doc):
    B = word.shape[0]
    fun = fun.astype(jnp.int32)
    word = word.astype(jnp.int32)
    context = context.astype(jnp.int32)
    neg = _neg_words(B)

    scores = _make_sc_scores(B)(W_hdn, W_vec, D_doc, neg, word, context, fun)
    return _loss_from_scores([scores], B)


# speedup vs baseline: 1.5150x; 1.4313x over previous
import functools

import numpy as np
import jax
import jax.numpy as jnp
from jax import lax
from jax.experimental import pallas as pl
from jax.experimental.pallas import tpu as pltpu
from jax.experimental.pallas import tpu_sc as plsc

_VOCAB = 100000
_N_NEGS = 64
_EMB = 128
_NCTX = 20
_NSAM = _N_NEGS + 1
_NSAM_PAD = 80
_NC = 2
_NS = 16
_NW = _NC * _NS
_NBUF = 2
_C8 = _EMB // 16

_neg_cache = {}


def _neg_words(batch):
    if batch in _neg_cache:
        return jnp.asarray(_neg_cache[batch])
    try:
        with jax.ensure_compile_time_eval():
            arr = np.asarray(
                jax.random.randint(jax.random.key(12345), (batch, _N_NEGS), 1, _VOCAB)
            )
        _neg_cache[batch] = arr
        return jnp.asarray(arr)
    except Exception:
        return jax.random.randint(jax.random.key(12345), (batch, _N_NEGS), 1, _VOCAB)


def _sc_body(whdn, wvec, ddoc, neg_idx, word_idx, ctx_idx, fun_idx, out,
             neg_idx_v, ctx_idx_v, word_idx_v, fun_idx_v, fun_rows, word_rows,
             sam_rows0, sam_rows1, ctx_rows0, ctx_rows1, scores_v,
             sem_f, sem_w, sem_s0, sem_s1, sem_c0, sem_c1):
    bt = scores_v.shape[0]
    sam_rows = (sam_rows0, sam_rows1)
    ctx_rows = (ctx_rows0, ctx_rows1)
    sem_s = (sem_s0, sem_s1)
    sem_c = (sem_c0, sem_c1)
    wid = lax.axis_index("s") * _NC + lax.axis_index("c")
    base = wid * bt

    pltpu.sync_copy(neg_idx.at[pl.ds(base, bt)], neg_idx_v)
    pltpu.sync_copy(ctx_idx.at[pl.ds(base, bt)], ctx_idx_v)
    pltpu.sync_copy(word_idx.at[pl.ds(base, bt)], word_idx_v)
    pltpu.sync_copy(fun_idx.at[pl.ds(base, bt)], fun_idx_v)
    fcopy = pltpu.async_copy(ddoc.at[fun_idx_v], fun_rows, sem_f)
    wcopy = pltpu.async_copy(whdn.at[word_idx_v], word_rows, sem_w)

    def fire(e, slot):
        pltpu.async_copy(whdn.at[neg_idx_v.at[e]], sam_rows[slot], sem_s[slot])
        pltpu.async_copy(wvec.at[ctx_idx_v.at[e]], ctx_rows[slot], sem_c[slot])

    def drain(slot):
        pltpu.make_async_copy(
            whdn.at[neg_idx_v.at[0]], sam_rows[slot], sem_s[slot]).wait()
        pltpu.make_async_copy(
            wvec.at[ctx_idx_v.at[0]], ctx_rows[slot], sem_c[slot]).wait()

    for slot in range(_NBUF):
        fire(slot, slot)
    fcopy.wait()
    wcopy.wait()

    lanes = lax.iota(jnp.int32, 16)

    def compute(e, slot):
        p = [fun_rows[e, pl.ds(16 * c, 16)] for c in range(_C8)]
        for j in range(_NCTX):
            for c in range(_C8):
                p[c] = p[c] + ctx_rows[slot][j, pl.ds(16 * c, 16)]
        inv = jnp.float32(1.0 / (_NCTX + 1))
        p = [x * inv for x in p]

        def acc_row(j):
            acc = sam_rows[slot][j, pl.ds(0, 16)] * p[0]
            for c in range(1, _C8):
                acc = acc + sam_rows[slot][j, pl.ds(16 * c, 16)] * p[c]
            return acc

        def grp_body(grp, carry):
            vec = jnp.zeros((16,), jnp.float32)
            for jj in range(16):
                vec = jnp.where(lanes == jj, jnp.sum(acc_row(grp * 16 + jj)), vec)
            scores_v[e, pl.ds(16 * grp, 16)] = -vec
            return carry

        lax.fori_loop(0, 4, grp_body, 0)
        wacc = word_rows[e, pl.ds(0, 16)] * p[0]
        for c in range(1, _C8):
            wacc = wacc + word_rows[e, pl.ds(16 * c, 16)] * p[c]
        wvecs = jnp.where(lanes == 0, jnp.sum(wacc), jnp.float32(0.0))
        scores_v[e, pl.ds(64, 16)] = wvecs

    def body(i, carry):
        for slot in range(_NBUF):
            e = i * _NBUF + slot
            drain(slot)
            compute(e, slot)
            fire(jnp.minimum(e + _NBUF, bt - 1), slot)
        return carry

    lax.fori_loop(0, bt // _NBUF, body, 0)
    for slot in range(_NBUF):
        drain(slot)
    pltpu.sync_copy(scores_v, out.at[pl.ds(base, bt)])


def _make_sc_scores(B):
    bt = B // _NW
    return pl.kernel(
        _sc_body,
        out_type=jax.ShapeDtypeStruct((B, _NSAM_PAD), jnp.float32),
        mesh=plsc.VectorSubcoreMesh(
            core_axis_name="c", subcore_axis_name="s",
            num_cores=_NC, num_subcores=_NS),
        compiler_params=pltpu.CompilerParams(needs_layout_passes=False),
        scratch_types=[
            pltpu.VMEM((bt, _N_NEGS), jnp.int32),
            pltpu.VMEM((bt, _NCTX), jnp.int32),
            pltpu.VMEM((bt,), jnp.int32),
            pltpu.VMEM((bt,), jnp.int32),
            pltpu.VMEM((bt, _EMB), jnp.float32),
            pltpu.VMEM((bt, _EMB), jnp.float32),
            pltpu.VMEM((_N_NEGS, _EMB), jnp.float32),
            pltpu.VMEM((_N_NEGS, _EMB), jnp.float32),
            pltpu.VMEM((_NCTX, _EMB), jnp.float32),
            pltpu.VMEM((_NCTX, _EMB), jnp.float32),
            pltpu.VMEM((bt, _NSAM_PAD), jnp.float32),
            pltpu.SemaphoreType.DMA,
            pltpu.SemaphoreType.DMA,
            pltpu.SemaphoreType.DMA,
            pltpu.SemaphoreType.DMA,
            pltpu.SemaphoreType.DMA,
            pltpu.SemaphoreType.DMA,
        ],
    )


def _neg_log_sig_sum(x):
    sim = jax.nn.sigmoid(x)
    masked = jnp.where(sim == 0.0, 1.0, sim)
    return -jnp.sum(jnp.log(masked))


def _loss_from_scores(scores_list, batch):
    def body(*refs):
        *s_refs, o_ref = refs
        total = jnp.float32(0.0)
        for s in s_refs:
            x = s[...]
            if x.shape[1] == _NSAM_PAD:
                x = x[:, :_NSAM]
            total = total + _neg_log_sig_sum(x)

        o_ref[...] = jnp.reshape(total / batch, (1, 1))

    out = pl.pallas_call(
        body,
        out_shape=jax.ShapeDtypeStruct((1, 1), jnp.float32),
    )(*scores_list)
    return out[0, 0]


def kernel(fun, word, context, W_hdn, W_vec, D_doc):
    B = word.shape[0]
    fun = fun.astype(jnp.int32)
    word = word.astype(jnp.int32)
    context = context.astype(jnp.int32)
    neg = _neg_words(B)

    scores = _make_sc_scores(B)(W_hdn, W_vec, D_doc, neg, word, context, fun)
    return _loss_from_scores([scores], B)


# speedup vs baseline: 1.6097x; 1.0625x over previous
import functools

import numpy as np
import jax
import jax.numpy as jnp
from jax import lax
from jax.experimental import pallas as pl
from jax.experimental.pallas import tpu as pltpu
from jax.experimental.pallas import tpu_sc as plsc

_VOCAB = 100000
_N_NEGS = 64
_EMB = 128
_NCTX = 20
_NSAM = _N_NEGS + 1
_NSAM_PAD = 80
_NC = 2
_NS = 16
_NW = _NC * _NS
_NBUF = 2
_C8 = _EMB // 16

_neg_cache = {}


def _neg_words(batch):
    if batch in _neg_cache:
        return jnp.asarray(_neg_cache[batch])
    try:
        with jax.ensure_compile_time_eval():
            arr = np.asarray(
                jax.random.randint(jax.random.key(12345), (batch, _N_NEGS), 1, _VOCAB)
            )
        _neg_cache[batch] = arr
        return jnp.asarray(arr)
    except Exception:
        return jax.random.randint(jax.random.key(12345), (batch, _N_NEGS), 1, _VOCAB)


def _sc_body(whdn, wvec, ddoc, neg_idx, word_idx, ctx_idx, fun_idx, out,
             neg_idx_v, ctx_idx_v, word_idx_v, fun_idx_v, fun_rows, word_rows,
             sam_rows0, sam_rows1, ctx_rows0, ctx_rows1, scores_v,
             sem_f, sem_w, sem_s0, sem_s1, sem_c0, sem_c1):
    bt = scores_v.shape[0]
    sam_rows = (sam_rows0, sam_rows1)
    ctx_rows = (ctx_rows0, ctx_rows1)
    sem_s = (sem_s0, sem_s1)
    sem_c = (sem_c0, sem_c1)
    wid = lax.axis_index("s") * _NC + lax.axis_index("c")
    base = wid * bt

    pltpu.sync_copy(neg_idx.at[pl.ds(base, bt)], neg_idx_v)
    pltpu.sync_copy(ctx_idx.at[pl.ds(base, bt)], ctx_idx_v)
    pltpu.sync_copy(word_idx.at[pl.ds(base, bt)], word_idx_v)
    pltpu.sync_copy(fun_idx.at[pl.ds(base, bt)], fun_idx_v)
    fcopy = pltpu.async_copy(ddoc.at[fun_idx_v], fun_rows, sem_f)
    wcopy = pltpu.async_copy(whdn.at[word_idx_v], word_rows, sem_w)

    def fire(e, slot):
        pltpu.async_copy(whdn.at[neg_idx_v.at[e]], sam_rows[slot], sem_s[slot])
        pltpu.async_copy(wvec.at[ctx_idx_v.at[e]], ctx_rows[slot], sem_c[slot])

    def drain(slot):
        pltpu.make_async_copy(
            whdn.at[neg_idx_v.at[0]], sam_rows[slot], sem_s[slot]).wait()
        pltpu.make_async_copy(
            wvec.at[ctx_idx_v.at[0]], ctx_rows[slot], sem_c[slot]).wait()

    for slot in range(_NBUF):
        fire(slot, slot)
    fcopy.wait()
    wcopy.wait()

    lanes = lax.iota(jnp.int32, 16)

    def compute(e, slot):
        p0 = [fun_rows[e, pl.ds(16 * c, 16)] for c in range(_C8)]

        def ctx_body(j, carry):
            return tuple(
                carry[c] + ctx_rows[slot][j, pl.ds(16 * c, 16)]
                for c in range(_C8))

        p = lax.fori_loop(0, _NCTX, ctx_body, tuple(p0))
        inv = jnp.float32(1.0 / (_NCTX + 1))
        p = [x * inv for x in p]

        def acc_row(j):
            acc = sam_rows[slot][j, pl.ds(0, 16)] * p[0]
            for c in range(1, _C8):
                acc = acc + sam_rows[slot][j, pl.ds(16 * c, 16)] * p[c]
            return acc

        def grp_body(grp, carry):
            vec = jnp.zeros((16,), jnp.float32)
            for jj in range(16):
                vec = jnp.where(lanes == jj, jnp.sum(acc_row(grp * 16 + jj)), vec)
            scores_v[e, pl.ds(16 * grp, 16)] = -vec
            return carry

        lax.fori_loop(0, 4, grp_body, 0)
        wacc = word_rows[e, pl.ds(0, 16)] * p[0]
        for c in range(1, _C8):
            wacc = wacc + word_rows[e, pl.ds(16 * c, 16)] * p[c]
        wvecs = jnp.where(lanes == 0, jnp.sum(wacc), jnp.float32(0.0))
        scores_v[e, pl.ds(64, 16)] = wvecs

    def body(i, carry):
        for slot in range(_NBUF):
            e = i * _NBUF + slot
            drain(slot)
            compute(e, slot)
            fire(jnp.minimum(e + _NBUF, bt - 1), slot)
        return carry

    lax.fori_loop(0, bt // _NBUF, body, 0)
    for slot in range(_NBUF):
        drain(slot)
    pltpu.sync_copy(scores_v, out.at[pl.ds(base, bt)])


def _make_sc_scores(B):
    bt = B // _NW
    return pl.kernel(
        _sc_body,
        out_type=jax.ShapeDtypeStruct((B, _NSAM_PAD), jnp.float32),
        mesh=plsc.VectorSubcoreMesh(
            core_axis_name="c", subcore_axis_name="s",
            num_cores=_NC, num_subcores=_NS),
        compiler_params=pltpu.CompilerParams(needs_layout_passes=False),
        scratch_types=[
            pltpu.VMEM((bt, _N_NEGS), jnp.int32),
            pltpu.VMEM((bt, _NCTX), jnp.int32),
            pltpu.VMEM((bt,), jnp.int32),
            pltpu.VMEM((bt,), jnp.int32),
            pltpu.VMEM((bt, _EMB), jnp.float32),
            pltpu.VMEM((bt, _EMB), jnp.float32),
            pltpu.VMEM((_N_NEGS, _EMB), jnp.float32),
            pltpu.VMEM((_N_NEGS, _EMB), jnp.float32),
            pltpu.VMEM((_NCTX, _EMB), jnp.float32),
            pltpu.VMEM((_NCTX, _EMB), jnp.float32),
            pltpu.VMEM((bt, _NSAM_PAD), jnp.float32),
            pltpu.SemaphoreType.DMA,
            pltpu.SemaphoreType.DMA,
            pltpu.SemaphoreType.DMA,
            pltpu.SemaphoreType.DMA,
            pltpu.SemaphoreType.DMA,
            pltpu.SemaphoreType.DMA,
        ],
    )


def _neg_log_sig_sum(x):
    sim = jax.nn.sigmoid(x)
    masked = jnp.where(sim == 0.0, 1.0, sim)
    return -jnp.sum(jnp.log(masked))


def _loss_from_scores(scores_list, batch):
    def body(*refs):
        *s_refs, o_ref = refs
        total = jnp.float32(0.0)
        for s in s_refs:
            x = s[...]
            if x.shape[1] == _NSAM_PAD:
                x = x[:, :_NSAM]
            total = total + _neg_log_sig_sum(x)

        o_ref[...] = jnp.reshape(total / batch, (1, 1))

    out = pl.pallas_call(
        body,
        out_shape=jax.ShapeDtypeStruct((1, 1), jnp.float32),
    )(*scores_list)
    return out[0, 0]


def kernel(fun, word, context, W_hdn, W_vec, D_doc):
    B = word.shape[0]
    fun = fun.astype(jnp.int32)
    word = word.astype(jnp.int32)
    context = context.astype(jnp.int32)
    neg = _neg_words(B)

    scores = _make_sc_scores(B)(W_hdn, W_vec, D_doc, neg, word, context, fun)
    return _loss_from_scores([scores], B)


# speedup vs baseline: 1.8009x; 1.1188x over previous
import functools

import numpy as np
import jax
import jax.numpy as jnp
from jax import lax
from jax.experimental import pallas as pl
from jax.experimental.pallas import tpu as pltpu
from jax.experimental.pallas import tpu_sc as plsc

_VOCAB = 100000
_N_NEGS = 64
_EMB = 128
_NCTX = 20
_NSAM = _N_NEGS + 1
_NSAM_PAD = 80
_NC = 2
_NS = 16
_NW = _NC * _NS
_NBUF = 4
_C8 = _EMB // 16

_neg_cache = {}


def _neg_words(batch):
    if batch in _neg_cache:
        return jnp.asarray(_neg_cache[batch])
    try:
        with jax.ensure_compile_time_eval():
            arr = np.asarray(
                jax.random.randint(jax.random.key(12345), (batch, _N_NEGS), 1, _VOCAB)
            )
        _neg_cache[batch] = arr
        return jnp.asarray(arr)
    except Exception:
        return jax.random.randint(jax.random.key(12345), (batch, _N_NEGS), 1, _VOCAB)


def _sc_body(whdn, wvec, ddoc, neg_idx, word_idx, ctx_idx, fun_idx, out,
             neg_idx_v, ctx_idx_v, word_idx_v, fun_idx_v, fun_rows, word_rows,
             sam_rows0, sam_rows1, sam_rows2, sam_rows3,
             ctx_rows0, ctx_rows1, ctx_rows2, ctx_rows3, scores_v,
             sem_f, sem_w, sem_s0, sem_s1, sem_s2, sem_s3,
             sem_c0, sem_c1, sem_c2, sem_c3):
    bt = scores_v.shape[0]
    sam_rows = (sam_rows0, sam_rows1, sam_rows2, sam_rows3)
    ctx_rows = (ctx_rows0, ctx_rows1, ctx_rows2, ctx_rows3)
    sem_s = (sem_s0, sem_s1, sem_s2, sem_s3)
    sem_c = (sem_c0, sem_c1, sem_c2, sem_c3)
    wid = lax.axis_index("s") * _NC + lax.axis_index("c")
    base = wid * bt

    pltpu.sync_copy(neg_idx.at[pl.ds(base, bt)], neg_idx_v)
    pltpu.sync_copy(ctx_idx.at[pl.ds(base, bt)], ctx_idx_v)
    pltpu.sync_copy(word_idx.at[pl.ds(base, bt)], word_idx_v)
    pltpu.sync_copy(fun_idx.at[pl.ds(base, bt)], fun_idx_v)
    fcopy = pltpu.async_copy(ddoc.at[fun_idx_v], fun_rows, sem_f)
    wcopy = pltpu.async_copy(whdn.at[word_idx_v], word_rows, sem_w)

    def fire(e, slot):
        pltpu.async_copy(whdn.at[neg_idx_v.at[e]], sam_rows[slot], sem_s[slot])
        pltpu.async_copy(wvec.at[ctx_idx_v.at[e]], ctx_rows[slot], sem_c[slot])

    def drain(slot):
        pltpu.make_async_copy(
            whdn.at[neg_idx_v.at[0]], sam_rows[slot], sem_s[slot]).wait()
        pltpu.make_async_copy(
            wvec.at[ctx_idx_v.at[0]], ctx_rows[slot], sem_c[slot]).wait()

    for slot in range(_NBUF):
        fire(slot, slot)
    fcopy.wait()
    wcopy.wait()

    lanes = lax.iota(jnp.int32, 16)

    def compute(e, slot):
        p0 = [fun_rows[e, pl.ds(16 * c, 16)] for c in range(_C8)]

        def ctx_body(j, carry):
            return tuple(
                carry[c] + ctx_rows[slot][j, pl.ds(16 * c, 16)]
                for c in range(_C8))

        p = lax.fori_loop(0, _NCTX, ctx_body, tuple(p0))
        inv = jnp.float32(1.0 / (_NCTX + 1))
        p = [x * inv for x in p]

        def acc_row(j):
            acc = sam_rows[slot][j, pl.ds(0, 16)] * p[0]
            for c in range(1, _C8):
                acc = acc + sam_rows[slot][j, pl.ds(16 * c, 16)] * p[c]
            return acc

        def grp_body(grp, carry):
            vec = jnp.zeros((16,), jnp.float32)
            for jj in range(16):
                vec = jnp.where(lanes == jj, jnp.sum(acc_row(grp * 16 + jj)), vec)
            scores_v[e, pl.ds(16 * grp, 16)] = -vec
            return carry

        lax.fori_loop(0, 4, grp_body, 0)
        wacc = word_rows[e, pl.ds(0, 16)] * p[0]
        for c in range(1, _C8):
            wacc = wacc + word_rows[e, pl.ds(16 * c, 16)] * p[c]
        wvecs = jnp.where(lanes == 0, jnp.sum(wacc), jnp.float32(0.0))
        scores_v[e, pl.ds(64, 16)] = wvecs

    def body(i, carry):
        for slot in range(_NBUF):
            e = i * _NBUF + slot
            drain(slot)
            compute(e, slot)
            fire(jnp.minimum(e + _NBUF, bt - 1), slot)
        return carry

    lax.fori_loop(0, bt // _NBUF, body, 0)
    for slot in range(_NBUF):
        drain(slot)
    pltpu.sync_copy(scores_v, out.at[pl.ds(base, bt)])


def _make_sc_scores(B):
    bt = B // _NW
    return pl.kernel(
        _sc_body,
        out_type=jax.ShapeDtypeStruct((B, _NSAM_PAD), jnp.float32),
        mesh=plsc.VectorSubcoreMesh(
            core_axis_name="c", subcore_axis_name="s",
            num_cores=_NC, num_subcores=_NS),
        compiler_params=pltpu.CompilerParams(needs_layout_passes=False),
        scratch_types=[
            pltpu.VMEM((bt, _N_NEGS), jnp.int32),
            pltpu.VMEM((bt, _NCTX), jnp.int32),
            pltpu.VMEM((bt,), jnp.int32),
            pltpu.VMEM((bt,), jnp.int32),
            pltpu.VMEM((bt, _EMB), jnp.float32),
            pltpu.VMEM((bt, _EMB), jnp.float32),
            pltpu.VMEM((_N_NEGS, _EMB), jnp.float32),
            pltpu.VMEM((_N_NEGS, _EMB), jnp.float32),
            pltpu.VMEM((_N_NEGS, _EMB), jnp.float32),
            pltpu.VMEM((_N_NEGS, _EMB), jnp.float32),
            pltpu.VMEM((_NCTX, _EMB), jnp.float32),
            pltpu.VMEM((_NCTX, _EMB), jnp.float32),
            pltpu.VMEM((_NCTX, _EMB), jnp.float32),
            pltpu.VMEM((_NCTX, _EMB), jnp.float32),
            pltpu.VMEM((bt, _NSAM_PAD), jnp.float32),
            pltpu.SemaphoreType.DMA,
            pltpu.SemaphoreType.DMA,
            pltpu.SemaphoreType.DMA,
            pltpu.SemaphoreType.DMA,
            pltpu.SemaphoreType.DMA,
            pltpu.SemaphoreType.DMA,
            pltpu.SemaphoreType.DMA,
            pltpu.SemaphoreType.DMA,
            pltpu.SemaphoreType.DMA,
            pltpu.SemaphoreType.DMA,
        ],
    )


def _neg_log_sig_sum(x):
    sim = jax.nn.sigmoid(x)
    masked = jnp.where(sim == 0.0, 1.0, sim)
    return -jnp.sum(jnp.log(masked))


def _loss_from_scores(scores_list, batch):
    def body(*refs):
        *s_refs, o_ref = refs
        total = jnp.float32(0.0)
        for s in s_refs:
            x = s[...]
            if x.shape[1] == _NSAM_PAD:
                x = x[:, :_NSAM]
            total = total + _neg_log_sig_sum(x)

        o_ref[...] = jnp.reshape(total / batch, (1, 1))

    out = pl.pallas_call(
        body,
        out_shape=jax.ShapeDtypeStruct((1, 1), jnp.float32),
    )(*scores_list)
    return out[0, 0]


def kernel(fun, word, context, W_hdn, W_vec, D_doc):
    B = word.shape[0]
    fun = fun.astype(jnp.int32)
    word = word.astype(jnp.int32)
    context = context.astype(jnp.int32)
    neg = _neg_words(B)

    scores = _make_sc_scores(B)(W_hdn, W_vec, D_doc, neg, word, context, fun)
    return _loss_from_scores([scores], B)
